# baseline (speedup 1.0000x reference)
import functools

import jax
import jax.numpy as jnp
from jax import lax
from jax.experimental import pallas as pl
from jax.experimental.pallas import tpu as pltpu
from jax.experimental.pallas import tpu_sc as plsc

D_MODEL = 64
SCALE = 8.0

NUM_CORES = 2
NUM_SUBCORES = 16
NUM_WORKERS = NUM_CORES * NUM_SUBCORES

LANE = 128
GROUP = 8
NBUF = 4


def _emb_body(xt_hbm, tp_hbm, out_hbm,
              raw, rows, tr, gsem, ssem, *, seq, na):
    wid = lax.axis_index("s") * NUM_CORES + lax.axis_index("c")
    a0 = wid * LANE

    iota = lax.iota(jnp.int32, 16)
    row_vs = [iota + (a8 * 16) for a8 in range(LANE // 16)]

    def stage_group(g):
        pltpu.sync_copy(xt_hbm.at[pl.ds(g * GROUP, GROUP), pl.ds(a0, LANE)],
                        raw.at[lax.rem(g, 2)])

    def fire(i, b):
        pltpu.async_copy(
            tp_hbm.at[raw.at[lax.rem(b // GROUP, 2), lax.rem(b, GROUP)]],
            rows.at[i], gsem)

    def wait_gather(i):
        pltpu.make_async_copy(tp_hbm.at[raw.at[0, 0]], rows.at[i],
                              gsem).wait()

    def transpose(i, j):
        @plsc.parallel_loop(0, LANE, unroll=2)
        def _(a):
            col_v = jnp.broadcast_to(a, (16,))
            for g in range(D_MODEL // 16):
                val = rows[i, a, pl.ds(g * 16, 16)]
                plsc.store_scatter(tr.at[j], [row_vs[g], col_v], val * SCALE)

    def store(j, b):
        pltpu.async_copy(tr.at[j, slice(None), pl.ds(0, LANE)],
                         out_hbm.at[b, slice(None), pl.ds(a0, LANE)],
                         ssem)

    def wait_store(j):
        pltpu.make_async_copy(tr.at[j, slice(None), pl.ds(0, LANE)],
                              out_hbm.at[0, slice(None), pl.ds(a0, LANE)],
                              ssem).wait()

    stage_group(0)
    for i in range(NBUF):
        fire(i, i)

    def step(b, carry):
        i = lax.rem(b, NBUF)
        j = lax.rem(b, 2)

        @pl.when((lax.rem(b, GROUP) == NBUF) & (b < seq - NBUF))
        def _():
            stage_group((b + NBUF) // GROUP)

        @pl.when(b >= 2)
        def _():
            wait_store(j)

        wait_gather(i)
        transpose(i, j)
        store(j, b)

        @pl.when(b < seq - NBUF)
        def _():
            fire(i, b + NBUF)

        return carry

    lax.fori_loop(0, seq, step, 0)
    wait_store(0)
    wait_store(1)


@jax.jit
def _emb(xt, tp):
    seq, na = xt.shape
    mesh = plsc.VectorSubcoreMesh(core_axis_name="c", subcore_axis_name="s")
    kern = pl.kernel(
        functools.partial(_emb_body, seq=seq, na=na),
        out_type=jax.ShapeDtypeStruct((seq, D_MODEL, na), jnp.float32),
        mesh=mesh,
        scratch_types=[
            pltpu.VMEM((2, GROUP, LANE), jnp.int32),
            pltpu.VMEM((NBUF, LANE, 2 * D_MODEL), jnp.float32),
            pltpu.VMEM((2, D_MODEL, LANE + 1), jnp.float32),
            pltpu.SemaphoreType.DMA,
            pltpu.SemaphoreType.DMA,
        ],
        compiler_params=pltpu.CompilerParams(use_tc_tiling_on_sc=True,
                                             needs_layout_passes=False),
    )
    return kern(xt, tp)


def kernel(x, table):
    na, seq = x.shape
    assert na == NUM_WORKERS * LANE and seq % GROUP == 0 and seq % NBUF == 0
    xt = jnp.transpose(x.astype(jnp.int32))
    tp = jnp.pad(table, ((0, 0), (0, 2 * D_MODEL - table.shape[1])))
    out_t = _emb(xt, tp)
    return jnp.transpose(out_t, (2, 0, 1))
